# baseline (speedup 1.0000x reference)
import functools

import jax
import jax.numpy as jnp
from jax import lax
from jax.experimental import pallas as pl
from jax.experimental.pallas import tpu as pltpu
from jax.experimental.pallas import tpu_sc as plsc

NC = 2
NS = 16
NW = NC * NS

CHUNK = 128


@functools.partial(jax.jit, static_argnames=("n_chunks",))
def _sc_gather(idx, table, n_chunks):
    E = table.shape[1]
    n_rows = idx.shape[0] * idx.shape[1] * idx.shape[2]

    mesh = plsc.VectorSubcoreMesh(core_axis_name="c", subcore_axis_name="s")

    @functools.partial(
        pl.kernel,
        mesh=mesh,
        out_type=jax.ShapeDtypeStruct((n_rows, E), jnp.float32),
        scratch_types=[
            pltpu.VMEM((n_chunks, CHUNK), jnp.int32),
            pltpu.VMEM((CHUNK, E), jnp.float32),
            pltpu.SemaphoreType.DMA,
        ],
        compiler_params=pltpu.CompilerParams(use_tc_tiling_on_sc=False),
    )
    def k(table_hbm, idx_hbm, out_hbm, idx_v, rows_v, sem):
        wid = lax.axis_index("s") * NC + lax.axis_index("c")
        base = wid * (n_chunks * CHUNK)
        pltpu.sync_copy(idx_hbm.at[wid], idx_v)

        def body(c, carry):
            pltpu.async_copy(table_hbm.at[idx_v.at[c]], rows_v, sem).wait()
            pltpu.sync_copy(rows_v, out_hbm.at[pl.ds(base + c * CHUNK, CHUNK)])
            return carry

        lax.fori_loop(0, n_chunks, body, 0)

    return k(table, idx)


def kernel(indices, table):
    B = indices.shape[0] * indices.shape[1]
    n_chunks = B // (NW * CHUNK)
    idx = indices.reshape(NW, n_chunks, CHUNK).astype(jnp.int32)
    out = _sc_gather(idx, table, n_chunks)
    return out.reshape(indices.shape[0], indices.shape[1], table.shape[1])


# speedup vs baseline: 2.2800x; 2.2800x over previous
import functools

import jax
import jax.numpy as jnp
from jax import lax
from jax.experimental import pallas as pl
from jax.experimental.pallas import tpu as pltpu
from jax.experimental.pallas import tpu_sc as plsc

NC = 2
NT = 16


def _sc_minor_gather(idx_flat, tabT, S, B, E, V):
    BPT = B // NT
    EPC = E // NC

    mesh = plsc.VectorSubcoreMesh(core_axis_name="c", subcore_axis_name="s")

    @functools.partial(
        pl.kernel,
        mesh=mesh,
        out_type=jax.ShapeDtypeStruct((S, E, B), jnp.float32),
        scratch_types=[
            pltpu.VMEM((S * BPT,), jnp.int32),
            pltpu.VMEM((S * BPT,), jnp.float32),
            pltpu.VMEM_SHARED((V,), jnp.float32),
            pltpu.SemaphoreType.DMA,
            pltpu.SemaphoreType.DMA,
            pltpu.SemaphoreType.DMA,
            pltpu.SemaphoreType.DMA,
        ],
    )
    def k(tabT_hbm, idx_hbm, out_hbm, idx_v, val_v, row_a,
          sem_idx, sem_row, sem_g, sem_o):
        c = lax.axis_index("c")
        t = lax.axis_index("s")
        e0 = c * EPC

        for s in range(S):
            pltpu.async_copy(
                idx_hbm.at[pl.ds(s * B + t * BPT, BPT)],
                idx_v.at[pl.ds(s * BPT, BPT)], sem_idx)
        for s in range(S):
            pltpu.make_async_copy(
                idx_hbm.at[pl.ds(s * B + t * BPT, BPT)],
                idx_v.at[pl.ds(s * BPT, BPT)], sem_idx,
            ).wait()

        @pl.when(t == 0)
        def _():
            pltpu.async_copy(tabT_hbm.at[e0], row_a, sem_row)

        def phase(e, row_cur):
            @pl.when(t == 0)
            def _():
                pltpu.make_async_copy(tabT_hbm.at[e], row_cur, sem_row).wait()
            plsc.subcore_barrier()

            for s in range(S):
                pltpu.async_copy(
                    row_cur.at[idx_v.at[pl.ds(s * BPT, BPT)]],
                    val_v.at[pl.ds(s * BPT, BPT)], sem_g)
            for s in range(S):
                pltpu.make_async_copy(
                    row_cur.at[idx_v.at[pl.ds(s * BPT, BPT)]],
                    val_v.at[pl.ds(s * BPT, BPT)], sem_g).wait()
            for s in range(S):
                pltpu.async_copy(
                    val_v.at[pl.ds(s * BPT, BPT)],
                    out_hbm.at[s, e, pl.ds(t * BPT, BPT)], sem_o)
            for s in range(S):
                pltpu.make_async_copy(
                    val_v.at[pl.ds(s * BPT, BPT)],
                    out_hbm.at[s, e, pl.ds(t * BPT, BPT)], sem_o,
                ).wait()

        def body(i, carry):
            e = e0 + i
            phase(e, row_a)
            plsc.subcore_barrier()

            @pl.when(jnp.logical_and(t == 0, i + 1 < EPC))
            def _():
                pltpu.async_copy(tabT_hbm.at[e + 1], row_a, sem_row)
            return carry

        lax.fori_loop(0, EPC, body, 0)

    return k(tabT, idx_flat)


def kernel(indices, table):
    B, S = indices.shape
    V, E = table.shape
    idx_flat = indices.T.astype(jnp.int32).reshape(S * B)
    tabT = table.T
    out3 = _sc_minor_gather(idx_flat, tabT, S, B, E, V)
    return out3.transpose(2, 0, 1)


# speedup vs baseline: 2.4007x; 1.0529x over previous
import functools

import jax
import jax.numpy as jnp
from jax import lax
from jax.experimental import pallas as pl
from jax.experimental.pallas import tpu as pltpu
from jax.experimental.pallas import tpu_sc as plsc

NC = 2
NT = 16


def _sc_minor_gather(idx_flat, tabT, tail2, S, B, E, V):
    BPT = B // NT
    EPC = E // NC
    CW = (V // NT) & ~127
    MAIN = NT * CW
    EXTRA = ((V - MAIN) // 128) * 128
    TB = V - MAIN - EXTRA
    VPAD = MAIN + EXTRA + 128 if TB else V

    mesh = plsc.VectorSubcoreMesh(core_axis_name="c", subcore_axis_name="s")

    @functools.partial(
        pl.kernel,
        mesh=mesh,
        out_type=jax.ShapeDtypeStruct((S, E, B), jnp.float32),
        scratch_types=[
            pltpu.VMEM((S * BPT,), jnp.int32),
            pltpu.VMEM((2 * S * BPT,), jnp.float32),
            pltpu.VMEM_SHARED((VPAD,), jnp.float32),
            pltpu.SemaphoreType.DMA,
            pltpu.SemaphoreType.DMA,
            pltpu.SemaphoreType.DMA,
            pltpu.SemaphoreType.DMA,
        ],
    )
    def k(tabT_hbm, idx_hbm, tail_hbm, out_hbm, idx_v, val_v, row_a,
          sem_idx, sem_row, sem_g, sem_o):
        c = lax.axis_index("c")
        t = lax.axis_index("s")
        e0 = c * EPC

        def stream_row(e):
            pltpu.async_copy(
                tabT_hbm.at[e].at[pl.ds(t * CW, CW)],
                row_a.at[pl.ds(t * CW, CW)], sem_row)
            if EXTRA:
                @pl.when(t == 0)
                def _():
                    pltpu.async_copy(
                        tabT_hbm.at[e].at[pl.ds(MAIN, EXTRA)],
                        row_a.at[pl.ds(MAIN, EXTRA)], sem_row)
            if TB:
                @pl.when(t == 1)
                def _():
                    pltpu.async_copy(
                        tail_hbm.at[pl.ds(e * 128, 128)],
                        row_a.at[pl.ds(MAIN + EXTRA, 128)], sem_row)

        def wait_row(e):
            pltpu.make_async_copy(
                tabT_hbm.at[e].at[pl.ds(t * CW, CW)],
                row_a.at[pl.ds(t * CW, CW)], sem_row).wait()
            if EXTRA:
                @pl.when(t == 0)
                def _():
                    pltpu.make_async_copy(
                        tabT_hbm.at[e].at[pl.ds(MAIN, EXTRA)],
                        row_a.at[pl.ds(MAIN, EXTRA)], sem_row).wait()
            if TB:
                @pl.when(t == 1)
                def _():
                    pltpu.make_async_copy(
                        tail_hbm.at[pl.ds(e * 128, 128)],
                        row_a.at[pl.ds(MAIN + EXTRA, 128)], sem_row).wait()

        def out_slot(par, s):
            return val_v.at[pl.ds((par * S + s) * BPT, BPT)]

        def drain_outs(par, e):
            for s in range(S):
                pltpu.make_async_copy(
                    out_slot(par, s),
                    out_hbm.at[s, e, pl.ds(t * BPT, BPT)], sem_o).wait()

        stream_row(e0)
        for s in range(S):
            pltpu.async_copy(
                idx_hbm.at[pl.ds(s * B + t * BPT, BPT)],
                idx_v.at[pl.ds(s * BPT, BPT)], sem_idx)
        for s in range(S):
            pltpu.make_async_copy(
                idx_hbm.at[pl.ds(s * B + t * BPT, BPT)],
                idx_v.at[pl.ds(s * BPT, BPT)], sem_idx).wait()

        def phase(i, e):
            par = e & 1
            wait_row(e)
            plsc.subcore_barrier()

            @pl.when(i >= 2)
            def _():
                drain_outs(par, e - 2)

            for s in range(S):
                pltpu.async_copy(
                    row_a.at[idx_v.at[pl.ds(s * BPT, BPT)]],
                    out_slot(par, s), sem_g)
            for s in range(S):
                pltpu.make_async_copy(
                    row_a.at[idx_v.at[pl.ds(s * BPT, BPT)]],
                    out_slot(par, s), sem_g).wait()
            for s in range(S):
                pltpu.async_copy(
                    out_slot(par, s),
                    out_hbm.at[s, e, pl.ds(t * BPT, BPT)], sem_o)

            plsc.subcore_barrier()

            @pl.when(i + 1 < EPC)
            def _():
                stream_row(e + 1)

        def body(i, carry):
            phase(i, e0 + i)
            return carry

        lax.fori_loop(0, EPC, body, 0)
        drain_outs((e0 + EPC - 2) & 1, e0 + EPC - 2)
        drain_outs((e0 + EPC - 1) & 1, e0 + EPC - 1)

    return k(tabT, idx_flat, tail2)


def kernel(indices, table):
    B, S = indices.shape
    V, E = table.shape
    idx_flat = indices.T.astype(jnp.int32).reshape(S * B)
    tabT = table.T
    nt = V % 128
    tail2 = jnp.pad(table[V - nt:].T, ((0, 0), (0, 128 - nt))).reshape(E * 128)
    out3 = _sc_minor_gather(idx_flat, tabT, tail2, S, B, E, V)
    return out3.transpose(2, 0, 1)


# speedup vs baseline: 2.4038x; 1.0013x over previous
import functools

import jax
import jax.numpy as jnp
from jax import lax
from jax.experimental import pallas as pl
from jax.experimental.pallas import tpu as pltpu
from jax.experimental.pallas import tpu_sc as plsc

NC = 2
NT = 16


def _sc_minor_gather(idx_flat, tabT, tail2, S, B, E, V):
    BPT = B // NT
    EPC = E // NC
    CW = (V // NT) & ~127
    MAIN = NT * CW
    EXTRA = ((V - MAIN) // 128) * 128
    TB = V - MAIN - EXTRA
    VPAD = MAIN + EXTRA + 128 if TB else V

    mesh = plsc.VectorSubcoreMesh(core_axis_name="c", subcore_axis_name="s")

    @functools.partial(
        pl.kernel,
        mesh=mesh,
        out_type=jax.ShapeDtypeStruct((S, E, B), jnp.float32),
        scratch_types=[
            pltpu.VMEM((S * BPT,), jnp.int32),
            pltpu.VMEM((2 * S * BPT,), jnp.float32),
            pltpu.VMEM_SHARED((VPAD,), jnp.float32),
            pltpu.SemaphoreType.DMA,
            pltpu.SemaphoreType.DMA,
            pltpu.SemaphoreType.DMA,
            pltpu.SemaphoreType.DMA,
        ],
    )
    def k(tabT_hbm, idx_hbm, tail_hbm, out_hbm, idx_v, val_v, row_a,
          sem_idx, sem_row, sem_g, sem_o):
        c = lax.axis_index("c")
        t = lax.axis_index("s")
        e0 = c * EPC

        def stream_row(e):
            pltpu.async_copy(
                tabT_hbm.at[e].at[pl.ds(t * CW, CW)],
                row_a.at[pl.ds(t * CW, CW)], sem_row)
            if EXTRA:
                @pl.when(t == 0)
                def _():
                    pltpu.async_copy(
                        tabT_hbm.at[e].at[pl.ds(MAIN, EXTRA)],
                        row_a.at[pl.ds(MAIN, EXTRA)], sem_row)
            if TB:
                @pl.when(t == 1)
                def _():
                    pltpu.async_copy(
                        tail_hbm.at[pl.ds(e * 128, 128)],
                        row_a.at[pl.ds(MAIN + EXTRA, 128)], sem_row)

        def wait_row(e):
            pltpu.make_async_copy(
                tabT_hbm.at[e].at[pl.ds(t * CW, CW)],
                row_a.at[pl.ds(t * CW, CW)], sem_row).wait()
            if EXTRA:
                @pl.when(t == 0)
                def _():
                    pltpu.make_async_copy(
                        tabT_hbm.at[e].at[pl.ds(MAIN, EXTRA)],
                        row_a.at[pl.ds(MAIN, EXTRA)], sem_row).wait()
            if TB:
                @pl.when(t == 1)
                def _():
                    pltpu.make_async_copy(
                        tail_hbm.at[pl.ds(e * 128, 128)],
                        row_a.at[pl.ds(MAIN + EXTRA, 128)], sem_row).wait()

        def out_slot(par, s):
            return val_v.at[pl.ds((par * S + s) * BPT, BPT)]

        def drain_outs(par, e):
            for s in range(S):
                pltpu.make_async_copy(
                    out_slot(par, s),
                    out_hbm.at[s, e, pl.ds(t * BPT, BPT)], sem_o).wait()

        stream_row(e0)
        for s in range(S):
            pltpu.async_copy(
                idx_hbm.at[pl.ds(s * B + t * BPT, BPT)],
                idx_v.at[pl.ds(s * BPT, BPT)], sem_idx)
        for s in range(S):
            pltpu.make_async_copy(
                idx_hbm.at[pl.ds(s * B + t * BPT, BPT)],
                idx_v.at[pl.ds(s * BPT, BPT)], sem_idx).wait()

        def phase(i, e):
            par = e & 1
            wait_row(e)
            plsc.subcore_barrier()

            @pl.when(i >= 2)
            def _():
                drain_outs(par, e - 2)

            pltpu.async_copy(
                row_a.at[idx_v], val_v.at[pl.ds(par * S * BPT, S * BPT)],
                sem_g)
            pltpu.make_async_copy(
                row_a.at[idx_v], val_v.at[pl.ds(par * S * BPT, S * BPT)],
                sem_g).wait()
            for s in range(S):
                pltpu.async_copy(
                    out_slot(par, s),
                    out_hbm.at[s, e, pl.ds(t * BPT, BPT)], sem_o)

            plsc.subcore_barrier()

            @pl.when(i + 1 < EPC)
            def _():
                stream_row(e + 1)

        def body(i, carry):
            phase(i, e0 + i)
            return carry

        lax.fori_loop(0, EPC, body, 0)
        drain_outs((e0 + EPC - 2) & 1, e0 + EPC - 2)
        drain_outs((e0 + EPC - 1) & 1, e0 + EPC - 1)

    return k(tabT, idx_flat, tail2)


def kernel(indices, table):
    B, S = indices.shape
    V, E = table.shape
    idx_flat = indices.T.astype(jnp.int32).reshape(S * B)
    tabT = table.T
    nt = V % 128
    tail2 = jnp.pad(table[V - nt:].T, ((0, 0), (0, 128 - nt))).reshape(E * 128)
    out3 = _sc_minor_gather(idx_flat, tabT, tail2, S, B, E, V)
    return out3.transpose(2, 0, 1)


# speedup vs baseline: 2.4625x; 1.0244x over previous
import functools

import jax
import jax.numpy as jnp
from jax import lax
from jax.experimental import pallas as pl
from jax.experimental.pallas import tpu as pltpu
from jax.experimental.pallas import tpu_sc as plsc

NC = 2
NT = 16


def _sc_minor_gather(idx_flat, tabT, tail2, S, B, E, V):
    BPT = B // NT
    EPC = E // NC
    MAIN = ((V // 128) // NT) * NT * 128
    EXTRA = ((V - MAIN) // 128) * 128
    TB = V - MAIN - EXTRA
    VP = MAIN + EXTRA + (128 if TB else 0)
    SCRATCH = NT * (S * BPT + S * BPT)
    X = ((2_097_024 - VP - SCRATCH) // 128) * 128

    mesh = plsc.VectorSubcoreMesh(core_axis_name="c", subcore_axis_name="s")

    @functools.partial(
        pl.kernel,
        mesh=mesh,
        out_type=jax.ShapeDtypeStruct((S, E, B), jnp.float32),
        scratch_types=[
            pltpu.VMEM((S * BPT,), jnp.int32),
            pltpu.VMEM((S * BPT,), jnp.float32),
            pltpu.VMEM_SHARED((VP + X,), jnp.float32),
            pltpu.SemaphoreType.DMA,
            pltpu.SemaphoreType.DMA,
            pltpu.SemaphoreType.DMA,
            pltpu.SemaphoreType.DMA,
        ],
    )
    def k(tabT_hbm, idx_hbm, tail_hbm, out_hbm, idx_v, val_v, row_buf,
          sem_idx, sem_row, sem_g, sem_o):
        c = lax.axis_index("c")
        t = lax.axis_index("s")
        e0 = c * EPC

        def span_copies(e, lo, hi, base):
            sz = hi - lo
            cw = ((sz // NT) // 128) * 128
            rem = sz - NT * cw
            cps = [(tabT_hbm.at[e].at[pl.ds(lo + t * cw, cw)],
                    row_buf.at[pl.ds(base + lo + t * cw, cw)], None)]
            if rem:
                cps.append((tabT_hbm.at[e].at[pl.ds(lo + NT * cw, rem)],
                            row_buf.at[pl.ds(base + lo + NT * cw, rem)], 0))
            return cps

        def end_copies(e, base):
            cps = []
            if EXTRA:
                cps.append((tabT_hbm.at[e].at[pl.ds(MAIN, EXTRA)],
                            row_buf.at[pl.ds(base + MAIN, EXTRA)], 1))
            if TB:
                cps.append((tail_hbm.at[pl.ds(e * 128, 128)],
                            row_buf.at[pl.ds(base + MAIN + EXTRA, 128)], 2))
            return cps

        def fire(cps):
            for src, dst, only in cps:
                if only is None:
                    pltpu.async_copy(src, dst, sem_row)
                else:
                    @pl.when(t == only)
                    def _():
                        pltpu.async_copy(src, dst, sem_row)

        def drain(cps):
            for src, dst, only in cps:
                if only is None:
                    pltpu.make_async_copy(src, dst, sem_row).wait()
                else:
                    @pl.when(t == only)
                    def _():
                        pltpu.make_async_copy(src, dst, sem_row).wait()

        def early(e, base):
            if base == 0:
                return span_copies(e, 0, X, 0)
            return span_copies(e, VP - X, MAIN, base) + end_copies(e, base)

        def late(e, base):
            if base == 0:
                return span_copies(e, X, MAIN, 0) + end_copies(e, 0)
            return span_copies(e, 0, VP - X, base)

        def out_slot(s):
            return val_v.at[pl.ds(s * BPT, BPT)]

        def drain_outs(e):
            for s in range(S):
                pltpu.make_async_copy(
                    out_slot(s),
                    out_hbm.at[s, e, pl.ds(t * BPT, BPT)], sem_o).wait()

        for s in range(S):
            pltpu.async_copy(
                idx_hbm.at[pl.ds(s * B + t * BPT, BPT)],
                idx_v.at[pl.ds(s * BPT, BPT)], sem_idx)

        fire(early(e0, X))
        fire(late(e0, X))
        fire(early(e0 + 1, 0))
        for s in range(S):
            pltpu.make_async_copy(
                idx_hbm.at[pl.ds(s * B + t * BPT, BPT)],
                idx_v.at[pl.ds(s * BPT, BPT)], sem_idx).wait()

        def phase(i, e, base):
            drain(early(e, base))
            drain(late(e, base))
            plsc.subcore_barrier()

            @pl.when(i >= 1)
            def _():
                drain_outs(e - 1)

            row_cur = row_buf.at[pl.ds(base, VP)]
            pltpu.async_copy(row_cur.at[idx_v], val_v, sem_g)
            pltpu.make_async_copy(row_cur.at[idx_v], val_v, sem_g).wait()
            for s in range(S):
                pltpu.async_copy(
                    out_slot(s),
                    out_hbm.at[s, e, pl.ds(t * BPT, BPT)], sem_o)

            plsc.subcore_barrier()

            oth = X - base
            @pl.when(i + 1 < EPC)
            def _():
                fire(late(e + 1, oth))
            @pl.when(i + 2 < EPC)
            def _():
                fire(early(e + 2, base))

        def body(j, carry):
            i = 2 * j
            e = e0 + i
            phase(i, e, X)
            phase(i + 1, e + 1, 0)
            return carry

        lax.fori_loop(0, EPC // 2, body, 0)
        drain_outs(e0 + EPC - 1)

    return k(tabT, idx_flat, tail2)


def kernel(indices, table):
    B, S = indices.shape
    V, E = table.shape
    idx_flat = indices.T.astype(jnp.int32).reshape(S * B)
    tabT = table.T
    nt = V % 128
    tail2 = jnp.pad(table[V - nt:].T, ((0, 0), (0, 128 - nt))).reshape(E * 128)
    out3 = _sc_minor_gather(idx_flat, tabT, tail2, S, B, E, V)
    return out3.transpose(2, 0, 1)


# speedup vs baseline: 2.4728x; 1.0042x over previous
import functools

import jax
import jax.numpy as jnp
from jax import lax
from jax.experimental import pallas as pl
from jax.experimental.pallas import tpu as pltpu
from jax.experimental.pallas import tpu_sc as plsc

NC = 2
NT = 16


def _sc_minor_gather(idx_flat, tabT, tail2, S, B, E, V):
    BPT = B // NT
    EPC = E // NC
    MAIN = ((V // 128) // NT) * NT * 128
    EXTRA = ((V - MAIN) // 128) * 128
    TB = V - MAIN - EXTRA
    VP = MAIN + EXTRA + (128 if TB else 0)
    SCRATCH = NT * (S * BPT + S * BPT)
    X = ((2_097_024 - VP - SCRATCH) // 128) * 128

    mesh = plsc.VectorSubcoreMesh(core_axis_name="c", subcore_axis_name="s")

    @functools.partial(
        pl.kernel,
        mesh=mesh,
        out_type=jax.ShapeDtypeStruct((S, E, B), jnp.float32),
        scratch_types=[
            pltpu.VMEM((S * BPT,), jnp.int32),
            pltpu.VMEM((S * BPT,), jnp.float32),
            pltpu.VMEM_SHARED((VP + X,), jnp.float32),
            pltpu.SemaphoreType.DMA,
            pltpu.SemaphoreType.DMA,
            pltpu.SemaphoreType.DMA,
            pltpu.SemaphoreType.DMA,
            pltpu.SemaphoreType.DMA,
        ],
    )
    def k(tabT_hbm, idx_hbm, tail_hbm, out_hbm, idx_v, val_v, row_buf,
          sem_idx, sem_row0, sem_row1, sem_g, sem_o):
        c = lax.axis_index("c")
        t = lax.axis_index("s")
        e0 = c * EPC

        def span_copies(e, lo, hi, base):
            sz = hi - lo
            cw = ((sz // NT) // 128) * 128
            rem = sz - NT * cw
            cps = [(tabT_hbm.at[e].at[pl.ds(lo + t * cw, cw)],
                    row_buf.at[pl.ds(base + lo + t * cw, cw)], None)]
            if rem:
                cps.append((tabT_hbm.at[e].at[pl.ds(lo + NT * cw, rem)],
                            row_buf.at[pl.ds(base + lo + NT * cw, rem)], 0))
            return cps

        def end_copies(e, base):
            cps = []
            if EXTRA:
                cps.append((tabT_hbm.at[e].at[pl.ds(MAIN, EXTRA)],
                            row_buf.at[pl.ds(base + MAIN, EXTRA)], 1))
            if TB:
                cps.append((tail_hbm.at[pl.ds(e * 128, 128)],
                            row_buf.at[pl.ds(base + MAIN + EXTRA, 128)], 2))
            return cps

        def fire(cps, par):
            sem = sem_row1 if par else sem_row0
            for src, dst, only in cps:
                if only is None:
                    pltpu.async_copy(src, dst, sem)
                else:
                    @pl.when(t == only)
                    def _():
                        pltpu.async_copy(src, dst, sem)

        def drain(cps, par):
            sem = sem_row1 if par else sem_row0
            for src, dst, only in cps:
                if only is None:
                    pltpu.make_async_copy(src, dst, sem).wait()
                else:
                    @pl.when(t == only)
                    def _():
                        pltpu.make_async_copy(src, dst, sem).wait()

        def early(e, base):
            if base == 0:
                return span_copies(e, 0, X, 0)
            return span_copies(e, VP - X, MAIN, base) + end_copies(e, base)

        def late(e, base):
            if base == 0:
                return span_copies(e, X, MAIN, 0) + end_copies(e, 0)
            return span_copies(e, 0, VP - X, base)

        def out_slot(s):
            return val_v.at[pl.ds(s * BPT, BPT)]

        def drain_outs(e):
            for s in range(S):
                pltpu.make_async_copy(
                    out_slot(s),
                    out_hbm.at[s, e, pl.ds(t * BPT, BPT)], sem_o).wait()

        for s in range(S):
            pltpu.async_copy(
                idx_hbm.at[pl.ds(s * B + t * BPT, BPT)],
                idx_v.at[pl.ds(s * BPT, BPT)], sem_idx)

        fire(early(e0, X), 0)
        fire(late(e0, X), 0)
        fire(early(e0 + 1, 0), 1)
        for s in range(S):
            pltpu.make_async_copy(
                idx_hbm.at[pl.ds(s * B + t * BPT, BPT)],
                idx_v.at[pl.ds(s * BPT, BPT)], sem_idx).wait()

        def phase(i, e, base, par):
            drain(early(e, base), par)
            drain(late(e, base), par)
            plsc.subcore_barrier()

            @pl.when(i >= 1)
            def _():
                drain_outs(e - 1)

            row_cur = row_buf.at[pl.ds(base, VP)]
            pltpu.async_copy(row_cur.at[idx_v], val_v, sem_g)
            pltpu.make_async_copy(row_cur.at[idx_v], val_v, sem_g).wait()
            for s in range(S):
                pltpu.async_copy(
                    out_slot(s),
                    out_hbm.at[s, e, pl.ds(t * BPT, BPT)], sem_o)

            plsc.subcore_barrier()

            oth = X - base
            @pl.when(i + 1 < EPC)
            def _():
                fire(late(e + 1, oth), 1 - par)
            @pl.when(i + 2 < EPC)
            def _():
                fire(early(e + 2, base), par)

        def body(j, carry):
            i = 2 * j
            e = e0 + i
            phase(i, e, X, 0)
            phase(i + 1, e + 1, 0, 1)
            return carry

        lax.fori_loop(0, EPC // 2, body, 0)
        drain_outs(e0 + EPC - 1)

    return k(tabT, idx_flat, tail2)


def kernel(indices, table):
    B, S = indices.shape
    V, E = table.shape
    idx_flat = indices.T.astype(jnp.int32).reshape(S * B)
    tabT = table.T
    nt = V % 128
    tail2 = jnp.pad(table[V - nt:].T, ((0, 0), (0, 128 - nt))).reshape(E * 128)
    out3 = _sc_minor_gather(idx_flat, tabT, tail2, S, B, E, V)
    return out3.transpose(2, 0, 1)
